# Initial kernel scaffold; baseline (speedup 1.0000x reference)
import jax
import jax.numpy as jnp
from jax.experimental import pallas as pl


def kernel(x, edge_index, branch_child, time_value, is_root, W1, b1, W2, b2, Wh1, bh1, Wh2, bh2, Wh3, bh3):
    raise NotImplementedError("write your pallas kernel here")


# speedup vs baseline: 15.0415x; 15.0415x over previous
import functools

import jax
import jax.numpy as jnp
from jax import lax
from jax.experimental import pallas as pl
from jax.experimental.pallas import tpu as pltpu
from jax.experimental.pallas import tpu_sc as plsc

N_NODES = 10000
N_LEAVES = 126
IN_DIM = N_LEAVES + 2
HID = 64
N_ACT = 4096
FOCAL = 5
N_REAL = N_NODES + 1
NPAD = 10240
N_EDGES = 320000
E_DIR = 2 * N_EDGES
NC, NS = 2, 16
NW = NC * NS
EK = 128
T_EDGE = 20480
E_PAD = NW * T_EDGE
N_ITER = T_EDGE // EK
DK = 2048
ROWS_PER_TILE = NPAD // NS

_mesh = plsc.VectorSubcoreMesh(core_axis_name="c", subcore_axis_name="s")
_sc_params = pltpu.CompilerParams(
    needs_layout_passes=False, use_tc_tiling_on_sc=False)


@functools.partial(
    pl.kernel,
    out_type=jax.ShapeDtypeStruct((NW, NPAD), jnp.float32),
    mesh=_mesh,
    compiler_params=_sc_params,
    scratch_types=[
        pltpu.VMEM((NPAD,), jnp.float32),
        pltpu.VMEM((DK,), jnp.int32),
    ],
)
def _sc_degree(dst_hbm, out_hbm, hist, ibuf):
    c = lax.axis_index("c")
    s = lax.axis_index("s")
    w = c * NS + s

    def _zero(i, _):
        hist[pl.ds(i * 16, 16)] = jnp.zeros((16,), jnp.float32)
        return 0
    lax.fori_loop(0, NPAD // 16, _zero, 0)

    ones16 = jnp.full((16,), 1.0, jnp.float32)

    def _chunk(i, _):
        pltpu.sync_copy(dst_hbm.at[pl.ds(w * T_EDGE + i * DK, DK)], ibuf)

        def _vec(j, _):
            idx = ibuf[pl.ds(j * 16, 16)]
            plsc.addupdate_scatter(hist, [idx], ones16)
            return 0
        lax.fori_loop(0, DK // 16, _vec, 0)
        return 0
    lax.fori_loop(0, T_EDGE // DK, _chunk, 0)

    pltpu.sync_copy(hist, out_hbm.at[w])


@functools.partial(
    pl.kernel,
    out_type=jax.ShapeDtypeStruct((NC * NPAD, HID), jnp.float32),
    mesh=_mesh,
    compiler_params=_sc_params,
    scratch_types=[
        pltpu.VMEM((EK,), jnp.int32),
        pltpu.VMEM((EK,), jnp.int32),
        pltpu.VMEM((EK, HID), jnp.float32),
        pltpu.VMEM((64, HID), jnp.float32),
        pltpu.VMEM_SHARED((NPAD, HID), jnp.float32),
        pltpu.SemaphoreType.DMA,
    ],
)
def _sc_scatter(g_hbm, src_hbm, dst_hbm, out_hbm, sidx, didx, rows, zbuf, acc, sem):
    c = lax.axis_index("c")
    s = lax.axis_index("s")
    w = c * NS + s

    def _zrow(i, _):
        for j in range(HID // 16):
            zbuf[i, pl.ds(j * 16, 16)] = jnp.zeros((16,), jnp.float32)
        return 0
    lax.fori_loop(0, 64, _zrow, 0)

    def _zacc(k, _):
        pltpu.sync_copy(zbuf, acc.at[pl.ds(s * ROWS_PER_TILE + k * 64, 64)])
        return 0
    lax.fori_loop(0, ROWS_PER_TILE // 64, _zacc, 0)

    plsc.subcore_barrier()

    def _edge(i, _):
        base = w * T_EDGE + i * EK
        pltpu.sync_copy(src_hbm.at[pl.ds(base, EK)], sidx)
        pltpu.sync_copy(dst_hbm.at[pl.ds(base, EK)], didx)
        pltpu.async_copy(g_hbm.at[sidx], rows, sem).wait()
        pltpu.sync_copy(rows, acc.at[didx], add=True)
        return 0
    lax.fori_loop(0, N_ITER, _edge, 0)

    plsc.subcore_barrier()
    pltpu.sync_copy(
        acc.at[pl.ds(s * ROWS_PER_TILE, ROWS_PER_TILE)],
        out_hbm.at[pl.ds(c * NPAD + s * ROWS_PER_TILE, ROWS_PER_TILE)],
    )


@functools.partial(
    pl.kernel,
    out_type=jax.ShapeDtypeStruct((N_ACT, HID), jnp.float32),
    mesh=_mesh,
    compiler_params=_sc_params,
    scratch_types=[
        pltpu.VMEM((N_ACT // NW,), jnp.int32),
        pltpu.VMEM((N_ACT // NW, HID), jnp.float32),
        pltpu.SemaphoreType.DMA,
    ],
)
def _sc_gather(emb_hbm, bc_hbm, out_hbm, ibuf, rows, sem):
    c = lax.axis_index("c")
    s = lax.axis_index("s")
    w = c * NS + s
    gk = N_ACT // NW
    pltpu.sync_copy(bc_hbm.at[pl.ds(w * gk, gk)], ibuf)
    pltpu.async_copy(emb_hbm.at[ibuf], rows, sem).wait()
    pltpu.sync_copy(rows, out_hbm.at[pl.ds(w * gk, gk)])


def _tc_prep_body(cnt_ref, xf_ref, w1_ref, dinv_ref, hw1_ref, g1_ref):
    cnt = cnt_ref[...]
    ones = jnp.ones((NW, 1), jnp.float32)
    deg = lax.dot_general(cnt, ones, (((0,), (0,)), ((), ())),
                          precision=lax.Precision.HIGHEST) + 1.0
    dinv = lax.rsqrt(deg)
    hw = jnp.dot(xf_ref[...], w1_ref[...],
                 precision=lax.Precision.HIGHEST)
    dinv_ref[...] = dinv
    hw1_ref[...] = hw
    g1_ref[...] = dinv * hw


def _tc_mid_body(aggp_ref, hw1_ref, dinv_ref, w2_ref, b1_ref,
                 hw2_ref, g2_ref):
    agg = aggp_ref[:NPAD, :] + aggp_ref[NPAD:, :]
    dinv = dinv_ref[...]
    h = jnp.maximum(dinv * agg + dinv * dinv * hw1_ref[...] + b1_ref[...], 0.0)
    hw2 = jnp.dot(h, w2_ref[...], precision=lax.Precision.HIGHEST)
    hw2_ref[...] = hw2
    g2_ref[...] = dinv * hw2


def _tc_emb_body(aggp_ref, hw2_ref, dinv_ref, b2_ref, emb_ref):
    agg = aggp_ref[:NPAD, :] + aggp_ref[NPAD:, :]
    dinv = dinv_ref[...]
    emb_ref[...] = dinv * agg + dinv * dinv * hw2_ref[...] + b2_ref[...]


def _elu(x):
    return jnp.where(x > 0.0, x, jnp.exp(x) - 1.0)


def _tc_head_body(ht_ref, hf_ref, t_ref, r_ref, wh1_ref, bh1_ref,
                  wh2_ref, bh2_ref, wh3_ref, bh3_ref,
                  ef_ref, logits_ref, probs_ref):
    ht = ht_ref[...]
    hf = jnp.broadcast_to(hf_ref[...], ht.shape)
    t = t_ref[...] * (1.0 / (1.0 + 1e-8))
    r = r_ref[...]
    ef = jnp.concatenate([hf, ht, jnp.abs(hf - ht), hf * ht, t, r], axis=1)
    ef_ref[...] = ef
    z = _elu(jnp.dot(ef, wh1_ref[...], precision=lax.Precision.HIGHEST)
             + bh1_ref[...])
    z = _elu(jnp.dot(z, wh2_ref[...], precision=lax.Precision.HIGHEST)
             + bh2_ref[...])
    lg = jnp.dot(z, wh3_ref[...], precision=lax.Precision.HIGHEST) + bh3_ref[...]
    m = jnp.max(lg)
    e = jnp.exp(lg - m)
    p = e / jnp.sum(e)
    logits_ref[...] = lg
    probs_ref[...] = p


_tc_prep = pl.pallas_call(
    _tc_prep_body,
    out_shape=[
        jax.ShapeDtypeStruct((NPAD, 1), jnp.float32),
        jax.ShapeDtypeStruct((NPAD, HID), jnp.float32),
        jax.ShapeDtypeStruct((NPAD, HID), jnp.float32),
    ],
)

_tc_mid = pl.pallas_call(
    _tc_mid_body,
    out_shape=[
        jax.ShapeDtypeStruct((NPAD, HID), jnp.float32),
        jax.ShapeDtypeStruct((NPAD, HID), jnp.float32),
    ],
)

_tc_emb = pl.pallas_call(
    _tc_emb_body,
    out_shape=jax.ShapeDtypeStruct((NPAD, HID), jnp.float32),
)

_tc_head = pl.pallas_call(
    _tc_head_body,
    out_shape=[
        jax.ShapeDtypeStruct((N_ACT, 4 * HID + 2), jnp.float32),
        jax.ShapeDtypeStruct((N_ACT, 1), jnp.float32),
        jax.ShapeDtypeStruct((N_ACT, 1), jnp.float32),
    ],
)


def kernel(x, edge_index, branch_child, time_value, is_root,
           W1, b1, W2, b2, Wh1, bh1, Wh2, bh2, Wh3, bh3):
    focal = (jnp.zeros((1, IN_DIM), jnp.float32)
             .at[0, FOCAL].set(1.0).at[0, N_LEAVES + 1].set(1.0))
    xf = jnp.concatenate(
        [x, focal, jnp.zeros((NPAD - N_REAL, IN_DIM), jnp.float32)], axis=0)

    e0 = edge_index[0].astype(jnp.int32)
    e1 = edge_index[1].astype(jnp.int32)
    pad = jnp.full((E_PAD - E_DIR,), NPAD - 1, jnp.int32)
    src = jnp.concatenate([e0, e1, pad])
    dst = jnp.concatenate([e1, e0, pad])

    cnt = _sc_degree(dst)
    dinv, hw1, g1 = _tc_prep(cnt, xf, W1)
    aggp1 = _sc_scatter(g1, src, dst)
    hw2, g2 = _tc_mid(aggp1, hw1, dinv, W2, b1.reshape(1, HID))
    aggp2 = _sc_scatter(g2, src, dst)
    emb_full = _tc_emb(aggp2, hw2, dinv, b2.reshape(1, HID))

    h_target = _sc_gather(emb_full, branch_child.astype(jnp.int32))
    hf = emb_full[N_REAL - 1:N_REAL]
    ef, logits, probs = _tc_head(
        h_target, hf, time_value.reshape(N_ACT, 1), is_root.reshape(N_ACT, 1),
        Wh1, bh1.reshape(1, HID), Wh2, bh2.reshape(1, HID),
        Wh3, bh3.reshape(1, 1))

    emb = emb_full[:N_REAL]
    leaf_feature = jnp.zeros((N_LEAVES,), jnp.float32).at[FOCAL].set(1.0)
    return logits[:, 0], probs[:, 0], ef, emb, leaf_feature


# speedup vs baseline: 19.0021x; 1.2633x over previous
import functools

import jax
import jax.numpy as jnp
from jax import lax
from jax.experimental import pallas as pl
from jax.experimental.pallas import tpu as pltpu
from jax.experimental.pallas import tpu_sc as plsc

N_NODES = 10000
N_LEAVES = 126
IN_DIM = N_LEAVES + 2
HID = 64
N_ACT = 4096
FOCAL = 5
N_REAL = N_NODES + 1
NPAD = 10240
N_EDGES = 320000
E_DIR = 2 * N_EDGES
NC, NS = 2, 16
NW = NC * NS
EK = 128
T_EDGE = 20480
E_PAD = NW * T_EDGE
N_ITER = T_EDGE // EK
DK = 2048
ROWS_PER_TILE = NPAD // NS

_mesh = plsc.VectorSubcoreMesh(core_axis_name="c", subcore_axis_name="s")
_sc_params = pltpu.CompilerParams(
    needs_layout_passes=False, use_tc_tiling_on_sc=False)


N_DCHUNK = T_EDGE // DK


@functools.partial(
    pl.kernel,
    out_type=jax.ShapeDtypeStruct((NW, NPAD), jnp.float32),
    mesh=_mesh,
    compiler_params=_sc_params,
    scratch_types=[
        pltpu.VMEM((NPAD,), jnp.float32),
        pltpu.VMEM((2, DK), jnp.int32),
        pltpu.SemaphoreType.DMA((2,)),
    ],
)
def _sc_degree(dst_hbm, out_hbm, hist, ibuf, dsem):
    c = lax.axis_index("c")
    s = lax.axis_index("s")
    w = c * NS + s

    def _zero(i, _):
        hist[pl.ds(i * 16, 16)] = jnp.zeros((16,), jnp.float32)
        return 0
    lax.fori_loop(0, NPAD // 16, _zero, 0)

    ones16 = jnp.full((16,), 1.0, jnp.float32)

    def _start(g, b):
        pltpu.async_copy(dst_hbm.at[pl.ds(w * T_EDGE + g * DK, DK)],
                         ibuf.at[b], dsem.at[b])

    def _wait(g, b):
        pltpu.make_async_copy(dst_hbm.at[pl.ds(w * T_EDGE + g * DK, DK)],
                              ibuf.at[b], dsem.at[b]).wait()

    def _process(b):
        def _vec(j, _):
            idx = ibuf[b, pl.ds(j * 16, 16)]
            plsc.addupdate_scatter(hist, [idx], ones16)
            return 0
        lax.fori_loop(0, DK // 16, _vec, 0)

    _start(0, 0)
    _start(1, 1)

    def _chunk(o, _):
        for b in range(2):
            g = o * 2 + b
            _wait(g, b)
            _process(b)
            _start(g + 2, b)
        return 0
    lax.fori_loop(0, (N_DCHUNK - 2) // 2, _chunk, 0)
    for g in (N_DCHUNK - 2, N_DCHUNK - 1):
        b = g % 2
        _wait(g, b)
        _process(b)

    pltpu.sync_copy(hist, out_hbm.at[w])


NBI = 8
NBR = 4


@functools.partial(
    pl.kernel,
    out_type=jax.ShapeDtypeStruct((NC * NPAD, HID), jnp.float32),
    mesh=_mesh,
    compiler_params=_sc_params,
    scratch_types=[
        pltpu.VMEM((NBI, 2, EK), jnp.int32),
        pltpu.VMEM((NBR, EK, HID), jnp.float32),
        pltpu.VMEM((64, HID), jnp.float32),
        pltpu.VMEM_SHARED((NPAD, HID), jnp.float32),
        pltpu.SemaphoreType.DMA((NBI,)),
        pltpu.SemaphoreType.DMA((NBR,)),
        pltpu.SemaphoreType.DMA((NBR,)),
    ],
)
def _sc_scatter(g_hbm, packed_hbm, out_hbm, eidx, rows, zbuf, acc,
                is_sem, gs_sem, ss_sem):
    c = lax.axis_index("c")
    s = lax.axis_index("s")
    w = c * NS + s
    base = w * N_ITER

    def _zrow(i, _):
        for j in range(HID // 16):
            zbuf[i, pl.ds(j * 16, 16)] = jnp.zeros((16,), jnp.float32)
        return 0
    lax.fori_loop(0, 64, _zrow, 0)

    def _zacc(k, _):
        pltpu.sync_copy(zbuf, acc.at[pl.ds(s * ROWS_PER_TILE + k * 64, 64)])
        return 0
    lax.fori_loop(0, ROWS_PER_TILE // 64, _zacc, 0)

    plsc.subcore_barrier()

    def idx_start(chunk, b):
        pltpu.async_copy(packed_hbm.at[base + chunk], eidx.at[b], is_sem.at[b])

    def idx_wait(chunk, b):
        pltpu.make_async_copy(
            packed_hbm.at[base + chunk], eidx.at[b], is_sem.at[b]).wait()

    def gather_start(b8, b4):
        pltpu.async_copy(g_hbm.at[eidx.at[b8, 0]], rows.at[b4], gs_sem.at[b4])

    def gather_wait(b8, b4):
        pltpu.make_async_copy(
            g_hbm.at[eidx.at[b8, 0]], rows.at[b4], gs_sem.at[b4]).wait()

    def scat_start(b8, b4):
        pltpu.async_copy(rows.at[b4], acc.at[eidx.at[b8, 1]], ss_sem.at[b4],
                         add=True)

    def scat_wait(b8, b4):
        pltpu.make_async_copy(
            rows.at[b4], acc.at[eidx.at[b8, 1]], ss_sem.at[b4]).wait()

    for p in range(5):
        idx_start(p, p)
    idx_wait(0, 0)
    gather_start(0, 0)
    for g in range(3):
        idx_wait(g + 1, g + 1)
        gather_start(g + 1, g + 1)
        gather_wait(g, g)
        scat_start(g, g)
        idx_start(g + 5, (g + 5) % NBI)

    def _body(o, _):
        for b in range(NBI):
            g = 3 + o * NBI + b
            scat_wait(b % NBI, b % NBR)
            idx_wait(g + 1, (4 + b) % NBI)
            gather_start((4 + b) % NBI, b % NBR)
            gather_wait((3 + b) % NBI, (3 + b) % NBR)
            scat_start((3 + b) % NBI, (3 + b) % NBR)
            idx_start(g + 5, b % NBI)
        return 0
    lax.fori_loop(0, (N_ITER - 8) // NBI, _body, 0)

    for g in range(N_ITER - 5, N_ITER - 1):
        scat_wait((g - 3) % NBI, (g - 3) % NBR)
        idx_wait(g + 1, (g + 1) % NBI)
        gather_start((g + 1) % NBI, (g + 1) % NBR)
        gather_wait(g % NBI, g % NBR)
        scat_start(g % NBI, g % NBR)
    g = N_ITER - 1
    scat_wait((g - 3) % NBI, (g - 3) % NBR)
    gather_wait(g % NBI, g % NBR)
    scat_start(g % NBI, g % NBR)
    for cch in range(N_ITER - 3, N_ITER):
        scat_wait(cch % NBI, cch % NBR)

    plsc.subcore_barrier()
    pltpu.sync_copy(
        acc.at[pl.ds(s * ROWS_PER_TILE, ROWS_PER_TILE)],
        out_hbm.at[pl.ds(c * NPAD + s * ROWS_PER_TILE, ROWS_PER_TILE)],
    )


@functools.partial(
    pl.kernel,
    out_type=jax.ShapeDtypeStruct((N_ACT, HID), jnp.float32),
    mesh=_mesh,
    compiler_params=_sc_params,
    scratch_types=[
        pltpu.VMEM((N_ACT // NW,), jnp.int32),
        pltpu.VMEM((N_ACT // NW, HID), jnp.float32),
        pltpu.SemaphoreType.DMA,
    ],
)
def _sc_gather(emb_hbm, bc_hbm, out_hbm, ibuf, rows, sem):
    c = lax.axis_index("c")
    s = lax.axis_index("s")
    w = c * NS + s
    gk = N_ACT // NW
    pltpu.sync_copy(bc_hbm.at[pl.ds(w * gk, gk)], ibuf)
    pltpu.async_copy(emb_hbm.at[ibuf], rows, sem).wait()
    pltpu.sync_copy(rows, out_hbm.at[pl.ds(w * gk, gk)])


def _tc_prep_body(cnt_ref, xf_ref, w1_ref, dinv_ref, hw1_ref, g1_ref):
    cnt = cnt_ref[...]
    ones = jnp.ones((NW, 1), jnp.float32)
    deg = lax.dot_general(cnt, ones, (((0,), (0,)), ((), ())),
                          precision=lax.Precision.HIGHEST) + 1.0
    dinv = lax.rsqrt(deg)
    hw = jnp.dot(xf_ref[...], w1_ref[...],
                 precision=lax.Precision.HIGHEST)
    dinv_ref[...] = dinv
    hw1_ref[...] = hw
    g1_ref[...] = dinv * hw


def _tc_mid_body(aggp_ref, hw1_ref, dinv_ref, w2_ref, b1_ref,
                 hw2_ref, g2_ref):
    agg = aggp_ref[:NPAD, :] + aggp_ref[NPAD:, :]
    dinv = dinv_ref[...]
    h = jnp.maximum(dinv * agg + dinv * dinv * hw1_ref[...] + b1_ref[...], 0.0)
    hw2 = jnp.dot(h, w2_ref[...], precision=lax.Precision.HIGHEST)
    hw2_ref[...] = hw2
    g2_ref[...] = dinv * hw2


def _tc_emb_body(aggp_ref, hw2_ref, dinv_ref, b2_ref, emb_ref):
    agg = aggp_ref[:NPAD, :] + aggp_ref[NPAD:, :]
    dinv = dinv_ref[...]
    emb_ref[...] = dinv * agg + dinv * dinv * hw2_ref[...] + b2_ref[...]


def _elu(x):
    return jnp.where(x > 0.0, x, jnp.exp(x) - 1.0)


def _tc_head_body(ht_ref, hf_ref, t_ref, r_ref, wh1_ref, bh1_ref,
                  wh2_ref, bh2_ref, wh3_ref, bh3_ref,
                  ef_ref, logits_ref, probs_ref):
    ht = ht_ref[...]
    hf = jnp.broadcast_to(hf_ref[...], ht.shape)
    t = t_ref[...] * (1.0 / (1.0 + 1e-8))
    r = r_ref[...]
    ef = jnp.concatenate([hf, ht, jnp.abs(hf - ht), hf * ht, t, r], axis=1)
    ef_ref[...] = ef
    z = _elu(jnp.dot(ef, wh1_ref[...], precision=lax.Precision.HIGHEST)
             + bh1_ref[...])
    z = _elu(jnp.dot(z, wh2_ref[...], precision=lax.Precision.HIGHEST)
             + bh2_ref[...])
    lg = jnp.dot(z, wh3_ref[...], precision=lax.Precision.HIGHEST) + bh3_ref[...]
    m = jnp.max(lg)
    e = jnp.exp(lg - m)
    p = e / jnp.sum(e)
    logits_ref[...] = lg
    probs_ref[...] = p


_tc_prep = pl.pallas_call(
    _tc_prep_body,
    out_shape=[
        jax.ShapeDtypeStruct((NPAD, 1), jnp.float32),
        jax.ShapeDtypeStruct((NPAD, HID), jnp.float32),
        jax.ShapeDtypeStruct((NPAD, HID), jnp.float32),
    ],
)

_tc_mid = pl.pallas_call(
    _tc_mid_body,
    out_shape=[
        jax.ShapeDtypeStruct((NPAD, HID), jnp.float32),
        jax.ShapeDtypeStruct((NPAD, HID), jnp.float32),
    ],
)

_tc_emb = pl.pallas_call(
    _tc_emb_body,
    out_shape=jax.ShapeDtypeStruct((NPAD, HID), jnp.float32),
)

_tc_head = pl.pallas_call(
    _tc_head_body,
    out_shape=[
        jax.ShapeDtypeStruct((N_ACT, 4 * HID + 2), jnp.float32),
        jax.ShapeDtypeStruct((N_ACT, 1), jnp.float32),
        jax.ShapeDtypeStruct((N_ACT, 1), jnp.float32),
    ],
)


def kernel(x, edge_index, branch_child, time_value, is_root,
           W1, b1, W2, b2, Wh1, bh1, Wh2, bh2, Wh3, bh3):
    focal = (jnp.zeros((1, IN_DIM), jnp.float32)
             .at[0, FOCAL].set(1.0).at[0, N_LEAVES + 1].set(1.0))
    xf = jnp.concatenate(
        [x, focal, jnp.zeros((NPAD - N_REAL, IN_DIM), jnp.float32)], axis=0)

    e0 = edge_index[0].astype(jnp.int32)
    e1 = edge_index[1].astype(jnp.int32)
    pad = jnp.full((E_PAD - E_DIR,), NPAD - 1, jnp.int32)
    src = jnp.concatenate([e0, e1, pad])
    dst = jnp.concatenate([e1, e0, pad])
    packed = jnp.stack(
        [src.reshape(E_PAD // EK, EK), dst.reshape(E_PAD // EK, EK)], axis=1)

    cnt = _sc_degree(dst)
    dinv, hw1, g1 = _tc_prep(cnt, xf, W1)
    aggp1 = _sc_scatter(g1, packed)
    hw2, g2 = _tc_mid(aggp1, hw1, dinv, W2, b1.reshape(1, HID))
    aggp2 = _sc_scatter(g2, packed)
    emb_full = _tc_emb(aggp2, hw2, dinv, b2.reshape(1, HID))

    h_target = _sc_gather(emb_full, branch_child.astype(jnp.int32))
    hf = emb_full[N_REAL - 1:N_REAL]
    ef, logits, probs = _tc_head(
        h_target, hf, time_value.reshape(N_ACT, 1), is_root.reshape(N_ACT, 1),
        Wh1, bh1.reshape(1, HID), Wh2, bh2.reshape(1, HID),
        Wh3, bh3.reshape(1, 1))

    emb = emb_full[:N_REAL]
    leaf_feature = jnp.zeros((N_LEAVES,), jnp.float32).at[FOCAL].set(1.0)
    return logits[:, 0], probs[:, 0], ef, emb, leaf_feature


# speedup vs baseline: 19.8472x; 1.0445x over previous
import functools

import jax
import jax.numpy as jnp
from jax import lax
from jax.experimental import pallas as pl
from jax.experimental.pallas import tpu as pltpu
from jax.experimental.pallas import tpu_sc as plsc

N_NODES = 10000
N_LEAVES = 126
IN_DIM = N_LEAVES + 2
HID = 64
N_ACT = 4096
FOCAL = 5
N_REAL = N_NODES + 1
NPAD = 10240
N_EDGES = 320000
E_DIR = 2 * N_EDGES
NC, NS = 2, 16
NW = NC * NS
EK = 128
T_EDGE = 20480
E_PAD = NW * T_EDGE
N_ITER = T_EDGE // EK
DK = 2048
ROWS_PER_TILE = NPAD // NS

_mesh = plsc.VectorSubcoreMesh(core_axis_name="c", subcore_axis_name="s")
_sc_params = pltpu.CompilerParams(
    needs_layout_passes=False, use_tc_tiling_on_sc=False)


N_DCHUNK = T_EDGE // DK


@functools.partial(
    pl.kernel,
    out_type=jax.ShapeDtypeStruct((NW, NPAD), jnp.float32),
    mesh=_mesh,
    compiler_params=_sc_params,
    scratch_types=[
        pltpu.VMEM((NPAD,), jnp.float32),
        pltpu.VMEM((2, DK), jnp.int32),
        pltpu.SemaphoreType.DMA((2,)),
    ],
)
def _sc_degree(dst_hbm, out_hbm, hist, ibuf, dsem):
    c = lax.axis_index("c")
    s = lax.axis_index("s")
    w = c * NS + s

    def _zero(i, _):
        hist[pl.ds(i * 16, 16)] = jnp.zeros((16,), jnp.float32)
        return 0
    lax.fori_loop(0, NPAD // 16, _zero, 0)

    ones16 = jnp.full((16,), 1.0, jnp.float32)

    def _start(g, b):
        pltpu.async_copy(dst_hbm.at[pl.ds(w * T_EDGE + g * DK, DK)],
                         ibuf.at[b], dsem.at[b])

    def _wait(g, b):
        pltpu.make_async_copy(dst_hbm.at[pl.ds(w * T_EDGE + g * DK, DK)],
                              ibuf.at[b], dsem.at[b]).wait()

    def _process(b):
        def _vec(j, _):
            idx = ibuf[b, pl.ds(j * 16, 16)]
            plsc.addupdate_scatter(hist, [idx], ones16)
            return 0
        lax.fori_loop(0, DK // 16, _vec, 0)

    _start(0, 0)
    _start(1, 1)

    def _chunk(o, _):
        for b in range(2):
            g = o * 2 + b
            _wait(g, b)
            _process(b)
            _start(g + 2, b)
        return 0
    lax.fori_loop(0, (N_DCHUNK - 2) // 2, _chunk, 0)
    for g in (N_DCHUNK - 2, N_DCHUNK - 1):
        b = g % 2
        _wait(g, b)
        _process(b)

    pltpu.sync_copy(hist, out_hbm.at[w])


NBI = 8
NBR = 4


@functools.partial(
    pl.kernel,
    out_type=jax.ShapeDtypeStruct((NC * NPAD, HID), jnp.float32),
    mesh=_mesh,
    compiler_params=_sc_params,
    scratch_types=[
        pltpu.VMEM((NBI, 2, EK), jnp.int32),
        pltpu.VMEM((NBR, EK, HID), jnp.float32),
        pltpu.VMEM((64, HID), jnp.float32),
        pltpu.VMEM_SHARED((NPAD, HID), jnp.float32),
        pltpu.SemaphoreType.DMA((NBI,)),
        pltpu.SemaphoreType.DMA((NBR,)),
        pltpu.SemaphoreType.DMA((NBR,)),
    ],
)
def _sc_scatter(g_hbm, packed_hbm, out_hbm, eidx, rows, zbuf, acc,
                is_sem, gs_sem, ss_sem):
    c = lax.axis_index("c")
    s = lax.axis_index("s")
    w = c * NS + s
    base = w * N_ITER

    def _zrow(i, _):
        for j in range(HID // 16):
            zbuf[i, pl.ds(j * 16, 16)] = jnp.zeros((16,), jnp.float32)
        return 0
    lax.fori_loop(0, 64, _zrow, 0)

    def _zacc(k, _):
        pltpu.sync_copy(zbuf, acc.at[pl.ds(s * ROWS_PER_TILE + k * 64, 64)])
        return 0
    lax.fori_loop(0, ROWS_PER_TILE // 64, _zacc, 0)

    plsc.subcore_barrier()

    def idx_start(chunk, b):
        pltpu.async_copy(packed_hbm.at[base + chunk], eidx.at[b], is_sem.at[b])

    def idx_wait(chunk, b):
        pltpu.make_async_copy(
            packed_hbm.at[base + chunk], eidx.at[b], is_sem.at[b]).wait()

    def gather_start(b8, b4):
        pltpu.async_copy(g_hbm.at[eidx.at[b8, 0]], rows.at[b4], gs_sem.at[b4])

    def gather_wait(b8, b4):
        pltpu.make_async_copy(
            g_hbm.at[eidx.at[b8, 0]], rows.at[b4], gs_sem.at[b4]).wait()

    def scat_start(b8, b4):
        pltpu.async_copy(rows.at[b4], acc.at[eidx.at[b8, 1]], ss_sem.at[b4],
                         add=True)

    def scat_wait(b8, b4):
        pltpu.make_async_copy(
            rows.at[b4], acc.at[eidx.at[b8, 1]], ss_sem.at[b4]).wait()

    for p in range(5):
        idx_start(p, p)
    idx_wait(0, 0)
    gather_start(0, 0)
    for g in range(3):
        idx_wait(g + 1, g + 1)
        gather_start(g + 1, g + 1)
        gather_wait(g, g)
        scat_start(g, g)
        idx_start(g + 5, (g + 5) % NBI)

    def _body(o, _):
        for b in range(NBI):
            g = 3 + o * NBI + b
            scat_wait(b % NBI, b % NBR)
            idx_wait(g + 1, (4 + b) % NBI)
            gather_start((4 + b) % NBI, b % NBR)
            gather_wait((3 + b) % NBI, (3 + b) % NBR)
            scat_start((3 + b) % NBI, (3 + b) % NBR)
            idx_start(g + 5, b % NBI)
        return 0
    lax.fori_loop(0, (N_ITER - 8) // NBI, _body, 0)

    for g in range(N_ITER - 5, N_ITER - 1):
        scat_wait((g - 3) % NBI, (g - 3) % NBR)
        idx_wait(g + 1, (g + 1) % NBI)
        gather_start((g + 1) % NBI, (g + 1) % NBR)
        gather_wait(g % NBI, g % NBR)
        scat_start(g % NBI, g % NBR)
    g = N_ITER - 1
    scat_wait((g - 3) % NBI, (g - 3) % NBR)
    gather_wait(g % NBI, g % NBR)
    scat_start(g % NBI, g % NBR)
    for cch in range(N_ITER - 3, N_ITER):
        scat_wait(cch % NBI, cch % NBR)

    plsc.subcore_barrier()
    pltpu.sync_copy(
        acc.at[pl.ds(s * ROWS_PER_TILE, ROWS_PER_TILE)],
        out_hbm.at[pl.ds(c * NPAD + s * ROWS_PER_TILE, ROWS_PER_TILE)],
    )


@functools.partial(
    pl.kernel,
    out_type=jax.ShapeDtypeStruct((N_ACT, HID), jnp.float32),
    mesh=_mesh,
    compiler_params=_sc_params,
    scratch_types=[
        pltpu.VMEM((N_ACT // NW,), jnp.int32),
        pltpu.VMEM((N_ACT // NW, HID), jnp.float32),
        pltpu.SemaphoreType.DMA,
    ],
)
def _sc_gather(emb_hbm, bc_hbm, out_hbm, ibuf, rows, sem):
    c = lax.axis_index("c")
    s = lax.axis_index("s")
    w = c * NS + s
    gk = N_ACT // NW
    pltpu.sync_copy(bc_hbm.at[pl.ds(w * gk, gk)], ibuf)
    pltpu.async_copy(emb_hbm.at[ibuf], rows, sem).wait()
    pltpu.sync_copy(rows, out_hbm.at[pl.ds(w * gk, gk)])


def _tc_prep_body(cnt_ref, xf_ref, w1_ref, dinv_ref, hw1_ref, g1_ref):
    cnt = cnt_ref[...]
    ones = jnp.ones((NW, 1), jnp.float32)
    deg = lax.dot_general(cnt, ones, (((0,), (0,)), ((), ())),
                          precision=lax.Precision.HIGHEST) + 1.0
    dinv = lax.rsqrt(deg)
    hw = jnp.dot(xf_ref[...], w1_ref[...],
                 precision=lax.Precision.HIGHEST)
    dinv_ref[...] = dinv
    hw1_ref[...] = hw
    g1_ref[...] = dinv * hw


def _tc_mid_body(aggp_ref, hw1_ref, dinv_ref, w2_ref, b1_ref,
                 hw2_ref, g2_ref):
    agg = aggp_ref[:NPAD, :] + aggp_ref[NPAD:, :]
    dinv = dinv_ref[...]
    h = jnp.maximum(dinv * agg + dinv * dinv * hw1_ref[...] + b1_ref[...], 0.0)
    hw2 = jnp.dot(h, w2_ref[...], precision=lax.Precision.HIGHEST)
    hw2_ref[...] = hw2
    g2_ref[...] = dinv * hw2


def _tc_emb_body(aggp_ref, hw2_ref, dinv_ref, b2_ref, emb_ref):
    agg = aggp_ref[:NPAD, :] + aggp_ref[NPAD:, :]
    dinv = dinv_ref[...]
    emb_ref[...] = dinv * agg + dinv * dinv * hw2_ref[...] + b2_ref[...]


def _elu(x):
    return jnp.where(x > 0.0, x, jnp.exp(x) - 1.0)


def _tc_head_body(ht_ref, hf_ref, t_ref, r_ref, wh1_ref, bh1_ref,
                  wh2_ref, bh2_ref, wh3_ref, bh3_ref,
                  ef_ref, logits_ref, probs_ref):
    ht = ht_ref[...]
    hf = jnp.broadcast_to(hf_ref[...], ht.shape)
    t = t_ref[...] * (1.0 / (1.0 + 1e-8))
    r = r_ref[...]
    ef = jnp.concatenate([hf, ht, jnp.abs(hf - ht), hf * ht, t, r], axis=1)
    ef_ref[...] = ef
    z = _elu(jnp.dot(ef, wh1_ref[...], precision=lax.Precision.HIGHEST)
             + bh1_ref[...])
    z = _elu(jnp.dot(z, wh2_ref[...], precision=lax.Precision.HIGHEST)
             + bh2_ref[...])
    lg = jnp.dot(z, wh3_ref[...], precision=lax.Precision.HIGHEST) + bh3_ref[...]
    m = jnp.max(lg)
    e = jnp.exp(lg - m)
    p = e / jnp.sum(e)
    logits_ref[...] = lg
    probs_ref[...] = p


_tc_prep = pl.pallas_call(
    _tc_prep_body,
    out_shape=[
        jax.ShapeDtypeStruct((NPAD, 1), jnp.float32),
        jax.ShapeDtypeStruct((NPAD, HID), jnp.float32),
        jax.ShapeDtypeStruct((NPAD, HID), jnp.float32),
    ],
)

_tc_mid = pl.pallas_call(
    _tc_mid_body,
    out_shape=[
        jax.ShapeDtypeStruct((NPAD, HID), jnp.float32),
        jax.ShapeDtypeStruct((NPAD, HID), jnp.float32),
    ],
)

_tc_emb = pl.pallas_call(
    _tc_emb_body,
    out_shape=jax.ShapeDtypeStruct((NPAD, HID), jnp.float32),
)

_tc_head = pl.pallas_call(
    _tc_head_body,
    out_shape=[
        jax.ShapeDtypeStruct((N_ACT, 4 * HID + 2), jnp.float32),
        jax.ShapeDtypeStruct((N_ACT, 1), jnp.float32),
        jax.ShapeDtypeStruct((N_ACT, 1), jnp.float32),
    ],
)


def kernel(x, edge_index, branch_child, time_value, is_root,
           W1, b1, W2, b2, Wh1, bh1, Wh2, bh2, Wh3, bh3):
    focal = (jnp.zeros((1, IN_DIM), jnp.float32)
             .at[0, FOCAL].set(1.0).at[0, N_LEAVES + 1].set(1.0))
    xf = jnp.concatenate(
        [x, focal, jnp.zeros((NPAD - N_REAL, IN_DIM), jnp.float32)], axis=0)

    e0 = edge_index[0].astype(jnp.int32)
    e1 = edge_index[1].astype(jnp.int32)
    pad_src = jnp.full((E_PAD - E_DIR,), NPAD - 1, jnp.int32)
    pad_dst = N_REAL + jnp.arange(E_PAD - E_DIR, dtype=jnp.int32) % (
        NPAD - N_REAL)
    src = jnp.concatenate([e0, e1, pad_src])
    dst = jnp.concatenate([e1, e0, pad_dst])
    packed = jnp.stack(
        [src.reshape(E_PAD // EK, EK), dst.reshape(E_PAD // EK, EK)], axis=1)

    cnt = _sc_degree(dst)
    dinv, hw1, g1 = _tc_prep(cnt, xf, W1)
    aggp1 = _sc_scatter(g1, packed)
    hw2, g2 = _tc_mid(aggp1, hw1, dinv, W2, b1.reshape(1, HID))
    aggp2 = _sc_scatter(g2, packed)
    emb_full = _tc_emb(aggp2, hw2, dinv, b2.reshape(1, HID))

    h_target = _sc_gather(emb_full, branch_child.astype(jnp.int32))
    hf = emb_full[N_REAL - 1:N_REAL]
    ef, logits, probs = _tc_head(
        h_target, hf, time_value.reshape(N_ACT, 1), is_root.reshape(N_ACT, 1),
        Wh1, bh1.reshape(1, HID), Wh2, bh2.reshape(1, HID),
        Wh3, bh3.reshape(1, 1))

    emb = emb_full[:N_REAL]
    leaf_feature = jnp.zeros((N_LEAVES,), jnp.float32).at[FOCAL].set(1.0)
    return logits[:, 0], probs[:, 0], ef, emb, leaf_feature


# speedup vs baseline: 20.8891x; 1.0525x over previous
import functools

import jax
import jax.numpy as jnp
from jax import lax
from jax.experimental import pallas as pl
from jax.experimental.pallas import tpu as pltpu
from jax.experimental.pallas import tpu_sc as plsc

N_NODES = 10000
N_LEAVES = 126
IN_DIM = N_LEAVES + 2
HID = 64
N_ACT = 4096
FOCAL = 5
N_REAL = N_NODES + 1
NPAD = 10240
N_EDGES = 320000
E_DIR = 2 * N_EDGES
NC, NS = 2, 16
NW = NC * NS
EK = 128
T_EDGE = 20480
E_PAD = NW * T_EDGE
N_ITER = T_EDGE // EK
DK = 2048
ROWS_PER_TILE = NPAD // NS

_mesh = plsc.VectorSubcoreMesh(core_axis_name="c", subcore_axis_name="s")
_sc_params = pltpu.CompilerParams(
    needs_layout_passes=False, use_tc_tiling_on_sc=False)


N_DCHUNK = T_EDGE // DK


@functools.partial(
    pl.kernel,
    out_type=jax.ShapeDtypeStruct((NW, NPAD), jnp.float32),
    mesh=_mesh,
    compiler_params=_sc_params,
    scratch_types=[
        pltpu.VMEM((NPAD,), jnp.float32),
        pltpu.VMEM((2, DK), jnp.int32),
        pltpu.SemaphoreType.DMA((2,)),
    ],
)
def _sc_degree(dst_hbm, out_hbm, hist, ibuf, dsem):
    c = lax.axis_index("c")
    s = lax.axis_index("s")
    w = c * NS + s

    def _zero(i, _):
        hist[pl.ds(i * 16, 16)] = jnp.zeros((16,), jnp.float32)
        return 0
    lax.fori_loop(0, NPAD // 16, _zero, 0)

    ones16 = jnp.full((16,), 1.0, jnp.float32)

    def _start(g, b):
        pltpu.async_copy(dst_hbm.at[pl.ds(w * T_EDGE + g * DK, DK)],
                         ibuf.at[b], dsem.at[b])

    def _wait(g, b):
        pltpu.make_async_copy(dst_hbm.at[pl.ds(w * T_EDGE + g * DK, DK)],
                              ibuf.at[b], dsem.at[b]).wait()

    def _process(b):
        def _vec(j, _):
            idx = ibuf[b, pl.ds(j * 16, 16)]
            plsc.addupdate_scatter(hist, [idx], ones16)
            return 0
        lax.fori_loop(0, DK // 16, _vec, 0)

    _start(0, 0)
    _start(1, 1)

    def _chunk(o, _):
        for b in range(2):
            g = o * 2 + b
            _wait(g, b)
            _process(b)
            _start(g + 2, b)
        return 0
    lax.fori_loop(0, (N_DCHUNK - 2) // 2, _chunk, 0)
    for g in (N_DCHUNK - 2, N_DCHUNK - 1):
        b = g % 2
        _wait(g, b)
        _process(b)

    pltpu.sync_copy(hist, out_hbm.at[w])


NBI = 8
NBR = 4


@functools.partial(
    pl.kernel,
    out_type=jax.ShapeDtypeStruct((NC * NPAD, HID), jnp.float32),
    mesh=_mesh,
    compiler_params=_sc_params,
    scratch_types=[
        pltpu.VMEM((NBI, 2, EK), jnp.int32),
        pltpu.VMEM((NBR, EK, HID), jnp.float32),
        pltpu.VMEM((64, HID), jnp.float32),
        pltpu.VMEM_SHARED((NPAD, HID), jnp.float32),
        pltpu.SemaphoreType.DMA((NBI,)),
        pltpu.SemaphoreType.DMA((NBR,)),
        pltpu.SemaphoreType.DMA((NBR,)),
    ],
)
def _sc_scatter(g_hbm, packed_hbm, out_hbm, eidx, rows, zbuf, acc,
                is_sem, gs_sem, ss_sem):
    c = lax.axis_index("c")
    s = lax.axis_index("s")
    w = c * NS + s
    base = w * N_ITER

    def _zrow(i, _):
        for j in range(HID // 16):
            zbuf[i, pl.ds(j * 16, 16)] = jnp.zeros((16,), jnp.float32)
        return 0
    lax.fori_loop(0, 64, _zrow, 0)

    def _zacc(k, _):
        pltpu.sync_copy(zbuf, acc.at[pl.ds(s * ROWS_PER_TILE + k * 64, 64)])
        return 0
    lax.fori_loop(0, ROWS_PER_TILE // 64, _zacc, 0)

    plsc.subcore_barrier()

    def idx_start(chunk, b):
        pltpu.async_copy(packed_hbm.at[base + chunk], eidx.at[b], is_sem.at[b])

    def idx_wait(chunk, b):
        pltpu.make_async_copy(
            packed_hbm.at[base + chunk], eidx.at[b], is_sem.at[b]).wait()

    def gather_start(b8, b4):
        pltpu.async_copy(g_hbm.at[eidx.at[b8, 0]], rows.at[b4], gs_sem.at[b4])

    def gather_wait(b8, b4):
        pltpu.make_async_copy(
            g_hbm.at[eidx.at[b8, 0]], rows.at[b4], gs_sem.at[b4]).wait()

    def scat_start(b8, b4):
        pltpu.async_copy(rows.at[b4], acc.at[eidx.at[b8, 1]], ss_sem.at[b4],
                         add=True)

    def scat_wait(b8, b4):
        pltpu.make_async_copy(
            rows.at[b4], acc.at[eidx.at[b8, 1]], ss_sem.at[b4]).wait()

    for p in range(5):
        idx_start(p, p)
    idx_wait(0, 0)
    gather_start(0, 0)
    for g in range(3):
        idx_wait(g + 1, g + 1)
        gather_start(g + 1, g + 1)
        gather_wait(g, g)
        scat_start(g, g)
        idx_start(g + 5, (g + 5) % NBI)

    def _body(o, _):
        for b in range(NBI):
            g = 3 + o * NBI + b
            scat_wait(b % NBI, b % NBR)
            idx_wait(g + 1, (4 + b) % NBI)
            gather_start((4 + b) % NBI, b % NBR)
            gather_wait((3 + b) % NBI, (3 + b) % NBR)
            scat_start((3 + b) % NBI, (3 + b) % NBR)
            idx_start(g + 5, b % NBI)
        return 0
    lax.fori_loop(0, (N_ITER - 8) // NBI, _body, 0)

    for g in range(N_ITER - 5, N_ITER - 1):
        scat_wait((g - 3) % NBI, (g - 3) % NBR)
        idx_wait(g + 1, (g + 1) % NBI)
        gather_start((g + 1) % NBI, (g + 1) % NBR)
        gather_wait(g % NBI, g % NBR)
        scat_start(g % NBI, g % NBR)
    g = N_ITER - 1
    scat_wait((g - 3) % NBI, (g - 3) % NBR)
    gather_wait(g % NBI, g % NBR)
    scat_start(g % NBI, g % NBR)
    for cch in range(N_ITER - 3, N_ITER):
        scat_wait(cch % NBI, cch % NBR)

    plsc.subcore_barrier()
    pltpu.sync_copy(
        acc.at[pl.ds(s * ROWS_PER_TILE, ROWS_PER_TILE)],
        out_hbm.at[pl.ds(c * NPAD + s * ROWS_PER_TILE, ROWS_PER_TILE)],
    )


@functools.partial(
    pl.kernel,
    out_type=jax.ShapeDtypeStruct((N_ACT, HID), jnp.float32),
    mesh=_mesh,
    compiler_params=_sc_params,
    scratch_types=[
        pltpu.VMEM((N_ACT // NW,), jnp.int32),
        pltpu.VMEM((N_ACT // NW, HID), jnp.float32),
        pltpu.SemaphoreType.DMA,
    ],
)
def _sc_gather(emb_hbm, bc_hbm, out_hbm, ibuf, rows, sem):
    c = lax.axis_index("c")
    s = lax.axis_index("s")
    w = c * NS + s
    gk = N_ACT // NW
    pltpu.sync_copy(bc_hbm.at[pl.ds(w * gk, gk)], ibuf)
    pltpu.async_copy(emb_hbm.at[ibuf], rows, sem).wait()
    pltpu.sync_copy(rows, out_hbm.at[pl.ds(w * gk, gk)])


def _tc_prep_body(cnt_ref, xf_ref, w1_ref, dinv_ref, hw1_ref, g1_ref):
    cnt = cnt_ref[...]
    ones = jnp.ones((NW, 1), jnp.float32)
    deg = lax.dot_general(cnt, ones, (((0,), (0,)), ((), ())),
                          precision=lax.Precision.HIGHEST) + 1.0
    y = lax.rsqrt(deg)
    y = y * (1.5 - 0.5 * deg * y * y)
    dinv = y * (1.5 - 0.5 * deg * y * y)
    hw = jnp.dot(xf_ref[...], w1_ref[...])
    dinv_ref[...] = dinv
    hw1_ref[...] = hw
    g1_ref[...] = dinv * hw


def _tc_mid_body(aggp_ref, hw1_ref, dinv_ref, w2_ref, b1_ref,
                 hw2_ref, g2_ref):
    agg = aggp_ref[:NPAD, :] + aggp_ref[NPAD:, :]
    dinv = dinv_ref[...]
    h = jnp.maximum(dinv * agg + dinv * dinv * hw1_ref[...] + b1_ref[...], 0.0)
    hw2 = jnp.dot(h, w2_ref[...])
    hw2_ref[...] = hw2
    g2_ref[...] = dinv * hw2


def _tc_emb_body(aggp_ref, hw2_ref, dinv_ref, b2_ref, emb_ref):
    agg = aggp_ref[:NPAD, :] + aggp_ref[NPAD:, :]
    dinv = dinv_ref[...]
    emb_ref[...] = dinv * agg + dinv * dinv * hw2_ref[...] + b2_ref[...]


def _elu(x):
    return jnp.where(x > 0.0, x, jnp.exp(x) - 1.0)


def _tc_head_body(ht_ref, hf_ref, t_ref, r_ref, wh1_ref, bh1_ref,
                  wh2_ref, bh2_ref, wh3_ref, bh3_ref,
                  ef_ref, logits_ref, probs_ref):
    ht = ht_ref[...]
    hf = jnp.broadcast_to(hf_ref[...], ht.shape)
    t = t_ref[...] * (1.0 / (1.0 + 1e-8))
    r = r_ref[...]
    ef = jnp.concatenate([hf, ht, jnp.abs(hf - ht), hf * ht, t, r], axis=1)
    ef_ref[...] = ef
    z = _elu(jnp.dot(ef, wh1_ref[...])
             + bh1_ref[...])
    z = _elu(jnp.dot(z, wh2_ref[...])
             + bh2_ref[...])
    lg = jnp.dot(z, wh3_ref[...]) + bh3_ref[...]
    m = jnp.max(lg)
    e = jnp.exp(lg - m)
    p = e / jnp.sum(e)
    logits_ref[...] = lg
    probs_ref[...] = p


_tc_prep = pl.pallas_call(
    _tc_prep_body,
    out_shape=[
        jax.ShapeDtypeStruct((NPAD, 1), jnp.float32),
        jax.ShapeDtypeStruct((NPAD, HID), jnp.float32),
        jax.ShapeDtypeStruct((NPAD, HID), jnp.float32),
    ],
)

_tc_mid = pl.pallas_call(
    _tc_mid_body,
    out_shape=[
        jax.ShapeDtypeStruct((NPAD, HID), jnp.float32),
        jax.ShapeDtypeStruct((NPAD, HID), jnp.float32),
    ],
)

_tc_emb = pl.pallas_call(
    _tc_emb_body,
    out_shape=jax.ShapeDtypeStruct((NPAD, HID), jnp.float32),
)

_tc_head = pl.pallas_call(
    _tc_head_body,
    out_shape=[
        jax.ShapeDtypeStruct((N_ACT, 4 * HID + 2), jnp.float32),
        jax.ShapeDtypeStruct((N_ACT, 1), jnp.float32),
        jax.ShapeDtypeStruct((N_ACT, 1), jnp.float32),
    ],
)


def kernel(x, edge_index, branch_child, time_value, is_root,
           W1, b1, W2, b2, Wh1, bh1, Wh2, bh2, Wh3, bh3):
    focal = (jnp.zeros((1, IN_DIM), jnp.float32)
             .at[0, FOCAL].set(1.0).at[0, N_LEAVES + 1].set(1.0))
    xf = jnp.concatenate(
        [x, focal, jnp.zeros((NPAD - N_REAL, IN_DIM), jnp.float32)], axis=0)

    e0 = edge_index[0].astype(jnp.int32)
    e1 = edge_index[1].astype(jnp.int32)
    pad_src = jnp.full((E_PAD - E_DIR,), NPAD - 1, jnp.int32)
    pad_dst = N_REAL + jnp.arange(E_PAD - E_DIR, dtype=jnp.int32) % (
        NPAD - N_REAL)
    src = jnp.concatenate([e0, e1, pad_src])
    dst = jnp.concatenate([e1, e0, pad_dst])
    packed = jnp.stack(
        [src.reshape(E_PAD // EK, EK), dst.reshape(E_PAD // EK, EK)], axis=1)

    cnt = _sc_degree(dst)
    dinv, hw1, g1 = _tc_prep(cnt, xf, W1)
    aggp1 = _sc_scatter(g1, packed)
    hw2, g2 = _tc_mid(aggp1, hw1, dinv, W2, b1.reshape(1, HID))
    aggp2 = _sc_scatter(g2, packed)
    emb_full = _tc_emb(aggp2, hw2, dinv, b2.reshape(1, HID))

    h_target = _sc_gather(emb_full, branch_child.astype(jnp.int32))
    hf = emb_full[N_REAL - 1:N_REAL]
    ef, logits, probs = _tc_head(
        h_target, hf, time_value.reshape(N_ACT, 1), is_root.reshape(N_ACT, 1),
        Wh1, bh1.reshape(1, HID), Wh2, bh2.reshape(1, HID),
        Wh3, bh3.reshape(1, 1))

    emb = emb_full[:N_REAL]
    leaf_feature = jnp.zeros((N_LEAVES,), jnp.float32).at[FOCAL].set(1.0)
    return logits[:, 0], probs[:, 0], ef, emb, leaf_feature


# speedup vs baseline: 22.1789x; 1.0617x over previous
import functools

import jax
import jax.numpy as jnp
from jax import lax
from jax.experimental import pallas as pl
from jax.experimental.pallas import tpu as pltpu
from jax.experimental.pallas import tpu_sc as plsc

N_NODES = 10000
N_LEAVES = 126
IN_DIM = N_LEAVES + 2
HID = 64
N_ACT = 4096
FOCAL = 5
N_REAL = N_NODES + 1
NPAD = 10240
N_EDGES = 320000
E_DIR = 2 * N_EDGES
NC, NS = 2, 16
NW = NC * NS
EK = 128
E_PAD = 655360
TOT_CHUNK = E_PAD // EK
ITER_F = 256
ITER_S = 64
FAST_C = 0
DK = 2048
ROWS_PER_TILE = NPAD // NS

_mesh = plsc.VectorSubcoreMesh(core_axis_name="c", subcore_axis_name="s")
_sc_params = pltpu.CompilerParams(
    needs_layout_passes=False, use_tc_tiling_on_sc=False)


E_FAST = ITER_F * EK
E_SLOW = ITER_S * EK


@functools.partial(
    pl.kernel,
    out_type=jax.ShapeDtypeStruct((NW, NPAD), jnp.float32),
    mesh=_mesh,
    compiler_params=_sc_params,
    scratch_types=[
        pltpu.VMEM((NPAD,), jnp.float32),
        pltpu.VMEM((2, DK), jnp.int32),
        pltpu.SemaphoreType.DMA((2,)),
    ],
)
def _sc_degree(dst_hbm, out_hbm, hist, ibuf, dsem):
    c = lax.axis_index("c")
    s = lax.axis_index("s")
    w = c * NS + s

    def _zero(i, _):
        hist[pl.ds(i * 16, 16)] = jnp.zeros((16,), jnp.float32)
        return 0
    lax.fori_loop(0, NPAD // 16, _zero, 0)

    ones16 = jnp.full((16,), 1.0, jnp.float32)

    def _dpipe(n_chunk, ebase):
        def _start(g, b):
            pltpu.async_copy(dst_hbm.at[pl.ds(ebase + g * DK, DK)],
                             ibuf.at[b], dsem.at[b])

        def _wait(g, b):
            pltpu.make_async_copy(dst_hbm.at[pl.ds(ebase + g * DK, DK)],
                                  ibuf.at[b], dsem.at[b]).wait()

        def _process(b):
            def _vec(j, _):
                idx = ibuf[b, pl.ds(j * 16, 16)]
                plsc.addupdate_scatter(hist, [idx], ones16)
                return 0
            lax.fori_loop(0, DK // 16, _vec, 0)

        _start(0, 0)
        _start(1, 1)

        def _chunk(o, _):
            for b in range(2):
                g = o * 2 + b
                _wait(g, b)
                _process(b)
                _start(g + 2, b)
            return 0
        lax.fori_loop(0, (n_chunk - 2) // 2, _chunk, 0)
        for g in (n_chunk - 2, n_chunk - 1):
            b = g % 2
            _wait(g, b)
            _process(b)

    @pl.when(c == FAST_C)
    def _():
        _dpipe(E_FAST // DK, s * E_FAST)

    @pl.when(c != FAST_C)
    def _():
        _dpipe(E_SLOW // DK, NS * E_FAST + s * E_SLOW)

    pltpu.sync_copy(hist, out_hbm.at[w])


NBI = 8
NBR = 4


@functools.partial(
    pl.kernel,
    out_type=jax.ShapeDtypeStruct((NC * NPAD, HID), jnp.float32),
    mesh=_mesh,
    compiler_params=_sc_params,
    scratch_types=[
        pltpu.VMEM((NBI, 2, EK), jnp.int32),
        pltpu.VMEM((NBR, EK, HID), jnp.float32),
        pltpu.VMEM((64, HID), jnp.float32),
        pltpu.VMEM_SHARED((NPAD, HID), jnp.float32),
        pltpu.SemaphoreType.DMA((NBI,)),
        pltpu.SemaphoreType.DMA((NBR,)),
        pltpu.SemaphoreType.DMA((NBR,)),
    ],
)
def _sc_scatter(g_hbm, packed_hbm, out_hbm, eidx, rows, zbuf, acc,
                is_sem, gs_sem, ss_sem):
    c = lax.axis_index("c")
    s = lax.axis_index("s")

    def _zrow(i, _):
        for j in range(HID // 16):
            zbuf[i, pl.ds(j * 16, 16)] = jnp.zeros((16,), jnp.float32)
        return 0
    lax.fori_loop(0, 64, _zrow, 0)

    def _zacc(k, _):
        pltpu.sync_copy(zbuf, acc.at[pl.ds(s * ROWS_PER_TILE + k * 64, 64)])
        return 0
    lax.fori_loop(0, ROWS_PER_TILE // 64, _zacc, 0)

    plsc.subcore_barrier()

    def _pipeline(n_iter, base):
        def idx_start(chunk, b):
            pltpu.async_copy(
                packed_hbm.at[base + chunk], eidx.at[b], is_sem.at[b])

        def idx_wait(chunk, b):
            pltpu.make_async_copy(
                packed_hbm.at[base + chunk], eidx.at[b], is_sem.at[b]).wait()

        def gather_start(b8, b4):
            pltpu.async_copy(
                g_hbm.at[eidx.at[b8, 0]], rows.at[b4], gs_sem.at[b4])

        def gather_wait(b8, b4):
            pltpu.make_async_copy(
                g_hbm.at[eidx.at[b8, 0]], rows.at[b4], gs_sem.at[b4]).wait()

        def scat_start(b8, b4):
            pltpu.async_copy(rows.at[b4], acc.at[eidx.at[b8, 1]],
                             ss_sem.at[b4], add=True)

        def scat_wait(b8, b4):
            pltpu.make_async_copy(
                rows.at[b4], acc.at[eidx.at[b8, 1]], ss_sem.at[b4]).wait()

        for p in range(5):
            idx_start(p, p)
        idx_wait(0, 0)
        gather_start(0, 0)
        for g in range(3):
            idx_wait(g + 1, g + 1)
            gather_start(g + 1, g + 1)
            gather_wait(g, g)
            scat_start(g, g)
            idx_start(g + 5, (g + 5) % NBI)

        def _body(o, _):
            for b in range(NBI):
                g = 3 + o * NBI + b
                scat_wait(b % NBI, b % NBR)
                idx_wait(g + 1, (4 + b) % NBI)
                gather_start((4 + b) % NBI, b % NBR)
                gather_wait((3 + b) % NBI, (3 + b) % NBR)
                scat_start((3 + b) % NBI, (3 + b) % NBR)
                idx_start(g + 5, b % NBI)
            return 0
        lax.fori_loop(0, (n_iter - 8) // NBI, _body, 0)

        for g in range(n_iter - 5, n_iter - 1):
            scat_wait((g - 3) % NBI, (g - 3) % NBR)
            idx_wait(g + 1, (g + 1) % NBI)
            gather_start((g + 1) % NBI, (g + 1) % NBR)
            gather_wait(g % NBI, g % NBR)
            scat_start(g % NBI, g % NBR)
        g = n_iter - 1
        scat_wait((g - 3) % NBI, (g - 3) % NBR)
        gather_wait(g % NBI, g % NBR)
        scat_start(g % NBI, g % NBR)
        for cch in range(n_iter - 3, n_iter):
            scat_wait(cch % NBI, cch % NBR)

    @pl.when(c == FAST_C)
    def _():
        _pipeline(ITER_F, s * ITER_F)

    @pl.when(c != FAST_C)
    def _():
        _pipeline(ITER_S, NS * ITER_F + s * ITER_S)

    plsc.subcore_barrier()
    pltpu.sync_copy(
        acc.at[pl.ds(s * ROWS_PER_TILE, ROWS_PER_TILE)],
        out_hbm.at[pl.ds(c * NPAD + s * ROWS_PER_TILE, ROWS_PER_TILE)],
    )


@functools.partial(
    pl.kernel,
    out_type=jax.ShapeDtypeStruct((N_ACT, HID), jnp.float32),
    mesh=_mesh,
    compiler_params=_sc_params,
    scratch_types=[
        pltpu.VMEM((N_ACT // NW,), jnp.int32),
        pltpu.VMEM((N_ACT // NW, HID), jnp.float32),
        pltpu.SemaphoreType.DMA,
    ],
)
def _sc_gather(emb_hbm, bc_hbm, out_hbm, ibuf, rows, sem):
    c = lax.axis_index("c")
    s = lax.axis_index("s")
    w = c * NS + s
    gk = N_ACT // NW
    pltpu.sync_copy(bc_hbm.at[pl.ds(w * gk, gk)], ibuf)
    pltpu.async_copy(emb_hbm.at[ibuf], rows, sem).wait()
    pltpu.sync_copy(rows, out_hbm.at[pl.ds(w * gk, gk)])


def _tc_prep_body(cnt_ref, xf_ref, w1_ref, dinv_ref, hw1_ref, g1_ref):
    cnt = cnt_ref[...]
    ones = jnp.ones((NW, 1), jnp.float32)
    deg = lax.dot_general(cnt, ones, (((0,), (0,)), ((), ())),
                          precision=lax.Precision.HIGHEST) + 1.0
    y = lax.rsqrt(deg)
    y = y * (1.5 - 0.5 * deg * y * y)
    dinv = y * (1.5 - 0.5 * deg * y * y)
    hw = jnp.dot(xf_ref[...], w1_ref[...])
    dinv_ref[...] = dinv
    hw1_ref[...] = hw
    g1_ref[...] = dinv * hw


def _tc_mid_body(aggp_ref, hw1_ref, dinv_ref, w2_ref, b1_ref,
                 hw2_ref, g2_ref):
    agg = aggp_ref[:NPAD, :] + aggp_ref[NPAD:, :]
    dinv = dinv_ref[...]
    h = jnp.maximum(dinv * agg + dinv * dinv * hw1_ref[...] + b1_ref[...], 0.0)
    hw2 = jnp.dot(h, w2_ref[...])
    hw2_ref[...] = hw2
    g2_ref[...] = dinv * hw2


def _tc_emb_body(aggp_ref, hw2_ref, dinv_ref, b2_ref, emb_ref):
    agg = aggp_ref[:NPAD, :] + aggp_ref[NPAD:, :]
    dinv = dinv_ref[...]
    emb_ref[...] = dinv * agg + dinv * dinv * hw2_ref[...] + b2_ref[...]


def _elu(x):
    return jnp.where(x > 0.0, x, jnp.exp(x) - 1.0)


def _tc_head_body(ht_ref, hf_ref, t_ref, r_ref, wh1_ref, bh1_ref,
                  wh2_ref, bh2_ref, wh3_ref, bh3_ref,
                  ef_ref, logits_ref, probs_ref):
    ht = ht_ref[...]
    hf = jnp.broadcast_to(hf_ref[...], ht.shape)
    t = t_ref[...] * (1.0 / (1.0 + 1e-8))
    r = r_ref[...]
    ef = jnp.concatenate([hf, ht, jnp.abs(hf - ht), hf * ht, t, r], axis=1)
    ef_ref[...] = ef
    z = _elu(jnp.dot(ef, wh1_ref[...])
             + bh1_ref[...])
    z = _elu(jnp.dot(z, wh2_ref[...])
             + bh2_ref[...])
    lg = jnp.dot(z, wh3_ref[...]) + bh3_ref[...]
    m = jnp.max(lg)
    e = jnp.exp(lg - m)
    p = e / jnp.sum(e)
    logits_ref[...] = lg
    probs_ref[...] = p


_tc_prep = pl.pallas_call(
    _tc_prep_body,
    out_shape=[
        jax.ShapeDtypeStruct((NPAD, 1), jnp.float32),
        jax.ShapeDtypeStruct((NPAD, HID), jnp.float32),
        jax.ShapeDtypeStruct((NPAD, HID), jnp.float32),
    ],
)

_tc_mid = pl.pallas_call(
    _tc_mid_body,
    out_shape=[
        jax.ShapeDtypeStruct((NPAD, HID), jnp.float32),
        jax.ShapeDtypeStruct((NPAD, HID), jnp.float32),
    ],
)

_tc_emb = pl.pallas_call(
    _tc_emb_body,
    out_shape=jax.ShapeDtypeStruct((NPAD, HID), jnp.float32),
)

_tc_head = pl.pallas_call(
    _tc_head_body,
    out_shape=[
        jax.ShapeDtypeStruct((N_ACT, 4 * HID + 2), jnp.float32),
        jax.ShapeDtypeStruct((N_ACT, 1), jnp.float32),
        jax.ShapeDtypeStruct((N_ACT, 1), jnp.float32),
    ],
)


def kernel(x, edge_index, branch_child, time_value, is_root,
           W1, b1, W2, b2, Wh1, bh1, Wh2, bh2, Wh3, bh3):
    focal = (jnp.zeros((1, IN_DIM), jnp.float32)
             .at[0, FOCAL].set(1.0).at[0, N_LEAVES + 1].set(1.0))
    xf = jnp.concatenate(
        [x, focal, jnp.zeros((NPAD - N_REAL, IN_DIM), jnp.float32)], axis=0)

    e0 = edge_index[0].astype(jnp.int32)
    e1 = edge_index[1].astype(jnp.int32)
    pad_src = jnp.full((E_PAD - E_DIR,), NPAD - 1, jnp.int32)
    pad_dst = N_REAL + jnp.arange(E_PAD - E_DIR, dtype=jnp.int32) % (
        NPAD - N_REAL)
    src = jnp.concatenate([e0, e1, pad_src])
    dst = jnp.concatenate([e1, e0, pad_dst])
    packed = jnp.stack(
        [src.reshape(E_PAD // EK, EK), dst.reshape(E_PAD // EK, EK)], axis=1)

    cnt = _sc_degree(dst)
    dinv, hw1, g1 = _tc_prep(cnt, xf, W1)
    aggp1 = _sc_scatter(g1, packed)
    hw2, g2 = _tc_mid(aggp1, hw1, dinv, W2, b1.reshape(1, HID))
    aggp2 = _sc_scatter(g2, packed)
    emb_full = _tc_emb(aggp2, hw2, dinv, b2.reshape(1, HID))

    h_target = _sc_gather(emb_full, branch_child.astype(jnp.int32))
    hf = emb_full[N_REAL - 1:N_REAL]
    ef, logits, probs = _tc_head(
        h_target, hf, time_value.reshape(N_ACT, 1), is_root.reshape(N_ACT, 1),
        Wh1, bh1.reshape(1, HID), Wh2, bh2.reshape(1, HID),
        Wh3, bh3.reshape(1, 1))

    emb = emb_full[:N_REAL]
    leaf_feature = jnp.zeros((N_LEAVES,), jnp.float32).at[FOCAL].set(1.0)
    return logits[:, 0], probs[:, 0], ef, emb, leaf_feature
